# baseline (speedup 1.0000x reference)
import functools

import jax
import jax.numpy as jnp
from jax.experimental import pallas as pl
from jax.experimental.pallas import tpu as pltpu

_WEIGHT = 0.01


def _chamfer_body(x_ref, y_ref, o_ref, xa_ref, ya_ref, rowacc_ref,
                  colmin_ref, *, n_blocks, m_blocks, tile_n, tile_m, n, m):
    nb = pl.program_id(1)
    mb = pl.program_id(2)

    ones8 = jnp.ones((128, 8), jnp.float32)

    @pl.when(mb == 0)
    def _():
        xf = x_ref[0]
        xa_ref[:, 0:128] = xf.astype(jnp.bfloat16)
        xa_ref[:, 128:129] = jnp.ones((tile_n, 1), jnp.bfloat16)
        x2 = jax.lax.dot_general(
            xf * xf, ones8, (((1,), (0,)), ((), ())),
            preferred_element_type=jnp.float32)[:, 0:1]
        xa_ref[:, 129:130] = x2.astype(jnp.bfloat16)

    @pl.when(nb == 0)
    def _():
        yf = y_ref[0]
        base = mb * tile_m
        ya_ref[pl.ds(base, tile_m), 0:128] = (-2.0 * yf).astype(jnp.bfloat16)
        y2 = jax.lax.dot_general(
            yf * yf, ones8, (((1,), (0,)), ((), ())),
            preferred_element_type=jnp.float32)[:, 0:1]
        ya_ref[pl.ds(base, tile_m), 128:129] = y2.astype(jnp.bfloat16)
        ya_ref[pl.ds(base, tile_m), 129:130] = jnp.ones((tile_m, 1),
                                                        jnp.bfloat16)

    xa = xa_ref[:, :]

    chunk = 512
    gm = None
    for c in range(tile_m // chunk):
        ya_c = ya_ref[pl.ds(mb * tile_m + c * chunk, chunk), :]
        d = jax.lax.dot_general(
            xa, ya_c, (((1,), (1,)), ((), ())),
            preferred_element_type=jnp.float32)

        bc = jnp.min(d, axis=0)
        sl_c = pl.ds(mb * tile_m + c * chunk, chunk)

        @pl.when(nb == 0)
        def _():
            colmin_ref[0, sl_c] = bc

        @pl.when(nb > 0)
        def _():
            colmin_ref[0, sl_c] = jnp.minimum(colmin_ref[0, sl_c], bc)

        for g in range(chunk // 128):
            part = d[:, g * 128:(g + 1) * 128]
            gm = part if gm is None else jnp.minimum(gm, part)

    @pl.when(jnp.logical_and(nb == 0, mb == 0))
    def _():
        o_ref[0, 0, :] = jnp.zeros((128,), jnp.float32)

    @pl.when(mb == 0)
    def _():
        rowacc_ref[:, :] = gm

    @pl.when(mb > 0)
    def _():
        rowacc_ref[:, :] = jnp.minimum(rowacc_ref[:, :], gm)

    @pl.when(mb == m_blocks - 1)
    def _():
        rowmin = jnp.min(rowacc_ref[:, :], axis=1)
        cham_x = jnp.maximum(rowmin, 0.0)
        o_ref[0, 0, :] += jnp.full((128,), jnp.sum(cham_x) * (1.0 / n))

    @pl.when(jnp.logical_and(nb == n_blocks - 1, mb == m_blocks - 1))
    def _():
        cham_y = jnp.maximum(colmin_ref[0, :], 0.0)
        o_ref[0, 0, :] += jnp.full((128,), jnp.sum(cham_y) * (1.0 / m))


def kernel(transformed_source, transformed_target):
    x = transformed_source.astype(jnp.float32)
    y = transformed_target.astype(jnp.float32)
    b, n, d = x.shape
    _, m, _ = y.shape

    tile_n = 2048
    tile_m = 2048
    n_blocks = n // tile_n
    m_blocks = m // tile_m

    body = functools.partial(
        _chamfer_body, n_blocks=n_blocks, m_blocks=m_blocks, tile_n=tile_n,
        tile_m=tile_m, n=n, m=m)

    out = pl.pallas_call(
        body,
        grid=(b, n_blocks, m_blocks),
        in_specs=[
            pl.BlockSpec((1, tile_n, d), lambda bi, ni, mi: (bi, ni, 0)),
            pl.BlockSpec((1, tile_m, d), lambda bi, ni, mi: (bi, mi, 0)),
        ],
        out_specs=pl.BlockSpec((1, 1, 128), lambda bi, ni, mi: (bi, 0, 0)),
        out_shape=jax.ShapeDtypeStruct((b, 1, 128), jnp.float32),
        scratch_shapes=[
            pltpu.VMEM((tile_n, d + 2), jnp.bfloat16),
            pltpu.VMEM((m, d + 2), jnp.bfloat16),
            pltpu.VMEM((tile_n, 128), jnp.float32),
            pltpu.VMEM((1, m), jnp.float32),
        ],
    )(x, y)

    return _WEIGHT * jnp.mean(out[:, 0, 0])
